# baseline (speedup 1.0000x reference)
import functools

import jax
import jax.numpy as jnp
from jax import lax
from jax.experimental import pallas as pl
from jax.experimental.pallas import tpu as pltpu
from jax.experimental.pallas import tpu_sc as plsc

_BBOX = 1000
_VOCAB = _BBOX + 3
_HID = 128
_L = 16
_C = 128
_H = _C // 2
_TPAD = 1024
_NT = 12

_SOURCES = [(0,), (1,), (2,), (3,), (4,), (5,), (6,), (7, 13), (8,),
            (9, 11), (10,), (12,)]


def _build(n_rows):
    info = plsc.get_sparse_core_info()
    nc, ns = info.num_cores, info.num_subcores
    nw = nc * ns
    rows_pw = n_rows // nw
    n_chunks = rows_pw // _C
    srows = _TPAD // ns
    mesh = plsc.VectorSubcoreMesh(core_axis_name="c", subcore_axis_name="s")

    @functools.partial(
        pl.kernel,
        mesh=mesh,
        out_type=jax.ShapeDtypeStruct((n_rows, _HID), jnp.float32),
        scratch_types=[
            pltpu.VMEM((2, 6, _C), jnp.int32),
            pltpu.VMEM((2, _NT, 2, _H), jnp.int32),
            pltpu.VMEM((_C, _HID), jnp.float32),
            pltpu.VMEM_SHARED((_NT * _TPAD, _HID), jnp.float32),
            pltpu.SemaphoreType.DMA,
            pltpu.SemaphoreType.DMA,
            pltpu.SemaphoreType.DMA,
            pltpu.SemaphoreType.DMA,
        ],
    )
    def k(comps_hbm, ctab_hbm, out_hbm, cv, idxv, acc, stab,
          gsem, psem, osem0, osem1):
        sid = lax.axis_index("s")
        wid = sid * nc + lax.axis_index("c")
        base0 = wid * rows_pw

        soff = sid * srows
        for e, srcs in enumerate(_SOURCES):
            dst = stab.at[pl.ds(e * _TPAD + soff, srows)]
            if len(srcs) == 1:
                pltpu.sync_copy(
                    ctab_hbm.at[pl.ds(srcs[0] * _TPAD + soff, srows)], dst)
            else:
                pltpu.sync_copy(
                    ctab_hbm.at[pl.ds(srcs[0] * _TPAD + soff, srows)],
                    acc.at[pl.ds(0, srows)])
                pltpu.sync_copy(
                    ctab_hbm.at[pl.ds(srcs[1] * _TPAD + soff, srows)],
                    acc.at[pl.ds(srows, srows)])

                def sum_body(r, carry):
                    for j in range(_HID // _L):
                        s = pl.ds(j * _L, _L)
                        acc[r, s] = acc[r, s] + acc[srows + r, s]
                    return carry

                lax.fori_loop(0, srows, sum_body, 0)
                pltpu.sync_copy(acc.at[pl.ds(0, srows)], dst)
        plsc.subcore_barrier()

        def comps_fetch(ci, pb):
            pltpu.async_copy(
                comps_hbm.at[:, pl.ds(base0 + ci * _C, _C)],
                cv.at[pb], psem)

        def comps_wait(ci, pb):
            pltpu.make_async_copy(
                comps_hbm.at[:, pl.ds(base0 + ci * _C, _C)],
                cv.at[pb], psem).wait()

        def idx_compute(pb):
            def idx_body(i, carry):
                s = i * _L
                cx = cv[pb, 0, pl.ds(s, _L)]
                cy = cv[pb, 1, pl.ds(s, _L)]
                w = cv[pb, 2, pl.ds(s, _L)]
                h = cv[pb, 3, pl.ds(s, _L)]
                xs = cv[pb, 4, pl.ds(s, _L)]
                ys = cv[pb, 5, pl.ds(s, _L)]
                xa = ((xs - _BBOX // 2).astype(jnp.float32) * 0.5
                      ).astype(jnp.int32)
                ya = ((ys - _BBOX // 2).astype(jnp.float32) * 0.5
                      ).astype(jnp.int32)
                w2 = lax.shift_right_arithmetic(w, 1)
                h2 = lax.shift_right_arithmetic(h, 1)

                def clip(v):
                    return jnp.minimum(jnp.maximum(v, 0), _BBOX)

                hh = i // (_H // _L)
                sl = pl.ds((s % _H), _L)
                idxv[pb, 0, hh, sl] = w
                idxv[pb, 1, hh, sl] = h + _TPAD
                idxv[pb, 2, hh, sl] = cx + 2 * _TPAD
                idxv[pb, 3, hh, sl] = cy + 3 * _TPAD
                idxv[pb, 4, hh, sl] = xs + 4 * _TPAD
                idxv[pb, 5, hh, sl] = ys + 5 * _TPAD
                idxv[pb, 6, hh, sl] = clip(cx - w2 - xa) + 6 * _TPAD
                idxv[pb, 7, hh, sl] = clip(cy - h2 - ya) + 7 * _TPAD
                idxv[pb, 8, hh, sl] = clip(cx + w2 - xa) + 8 * _TPAD
                idxv[pb, 9, hh, sl] = clip(cy + h2 + ya) + 9 * _TPAD
                idxv[pb, 10, hh, sl] = clip(cx + w2 + xa) + 10 * _TPAD
                idxv[pb, 11, hh, sl] = clip(cx - w2 + xa) + 11 * _TPAD
                return carry

            lax.fori_loop(0, _C // _L, idx_body, 0)

        def zero_half(hh):
            def zero_body(r, carry):
                for j in range(_HID // _L):
                    acc[r, pl.ds(j * _L, _L)] = jnp.zeros((_L,), jnp.float32)
                return carry
            lax.fori_loop(hh * _H, hh * _H + _H, zero_body, 0)

        def fire_half(pb, hh):
            def fire_body(t, carry):
                pltpu.async_copy(stab.at[idxv.at[pb, t, hh]],
                                 acc.at[pl.ds(hh * _H, _H)], gsem, add=True)
                return carry
            lax.fori_loop(0, _NT, fire_body, 0)

        def drain_half(pb, hh):
            def drain_body(t, carry):
                pltpu.make_async_copy(stab.at[idxv.at[pb, t, hh]],
                                     acc.at[pl.ds(hh * _H, _H)], gsem).wait()
                return carry
            lax.fori_loop(0, _NT, drain_body, 0)

        def out_start(ci, hh, osem):
            sl = pl.ds(base0 + ci * _C + hh * _H, _H)
            pltpu.async_copy(acc.at[pl.ds(hh * _H, _H)], out_hbm.at[sl], osem)

        def out_wait(ci, hh, osem):
            sl = pl.ds(base0 + ci * _C + hh * _H, _H)
            pltpu.make_async_copy(acc.at[pl.ds(hh * _H, _H)],
                                  out_hbm.at[sl], osem).wait()

        comps_fetch(0, 0)
        comps_wait(0, 0)
        idx_compute(0)
        comps_fetch(1, 1)

        def chunk_body(ci, carry):
            pb = lax.rem(ci, 2)

            @pl.when(ci > 0)
            def _wait_prev_out0():
                out_wait(ci - 1, 0, osem0)

            zero_half(0)
            fire_half(pb, 0)

            @pl.when(ci > 0)
            def _wait_prev_out1():
                out_wait(ci - 1, 1, osem1)

            zero_half(1)
            fire_half(pb, 1)

            @pl.when(ci + 2 < n_chunks)
            def _prefetch():
                comps_fetch(ci + 2, pb)

            @pl.when(ci + 1 < n_chunks)
            def _idx_next():
                comps_wait(ci + 1, 1 - pb)
                idx_compute(1 - pb)

            drain_half(pb, 0)
            out_start(ci, 0, osem0)
            drain_half(pb, 1)
            out_start(ci, 1, osem1)
            return carry

        lax.fori_loop(0, n_chunks, chunk_body, 0)
        out_wait(n_chunks - 1, 0, osem0)
        out_wait(n_chunks - 1, 1, osem1)

    return k


def kernel(boxes, tables):
    b, s, _ = boxes.shape
    n_rows = b * s
    comps = boxes.astype(jnp.int32).reshape(n_rows, 6).T
    ctab = jnp.pad(tables, ((0, 0), (0, _TPAD - _VOCAB), (0, 0))
                   ).reshape(14 * _TPAD, _HID)
    out = _build(n_rows)(comps, ctab)
    return out.reshape(b, s, _HID)


# speedup vs baseline: 1.0012x; 1.0012x over previous
import functools

import jax
import jax.numpy as jnp
from jax import lax
from jax.experimental import pallas as pl
from jax.experimental.pallas import tpu as pltpu
from jax.experimental.pallas import tpu_sc as plsc

_BBOX = 1000
_VOCAB = _BBOX + 3
_HID = 128
_L = 16
_C = 128
_TPAD = 1024
_NT = 12

_SOURCES = [(0,), (1,), (2,), (3,), (4,), (5,), (6,), (7, 13), (8,),
            (9, 11), (10,), (12,)]


def _build(n_rows):
    info = plsc.get_sparse_core_info()
    nc, ns = info.num_cores, info.num_subcores
    nw = nc * ns
    rows_pw = n_rows // nw
    n_chunks = rows_pw // _C
    srows = _TPAD // ns
    mesh = plsc.VectorSubcoreMesh(core_axis_name="c", subcore_axis_name="s")

    @functools.partial(
        pl.kernel,
        mesh=mesh,
        out_type=jax.ShapeDtypeStruct((n_rows, _HID), jnp.float32),
        scratch_types=[
            pltpu.VMEM((6, _C), jnp.int32),
            pltpu.VMEM((_NT, 1, _C), jnp.int32),
            pltpu.VMEM((_C, _HID), jnp.float32),
            pltpu.VMEM_SHARED((_NT * _TPAD, _HID), jnp.float32),
            pltpu.SemaphoreType.DMA,
        ],
    )
    def k(comps_hbm, ctab_hbm, out_hbm, cv, idxv, acc, stab, gsem):
        sid = lax.axis_index("s")
        wid = sid * nc + lax.axis_index("c")
        base0 = wid * rows_pw

        soff = sid * srows
        for e, srcs in enumerate(_SOURCES):
            dst = stab.at[pl.ds(e * _TPAD + soff, srows)]
            if len(srcs) == 1:
                pltpu.sync_copy(
                    ctab_hbm.at[pl.ds(srcs[0] * _TPAD + soff, srows)], dst)
            else:
                pltpu.sync_copy(
                    ctab_hbm.at[pl.ds(srcs[0] * _TPAD + soff, srows)],
                    acc.at[pl.ds(0, srows)])
                pltpu.sync_copy(
                    ctab_hbm.at[pl.ds(srcs[1] * _TPAD + soff, srows)],
                    acc.at[pl.ds(srows, srows)])

                def sum_body(r, carry):
                    for j in range(_HID // _L):
                        s = pl.ds(j * _L, _L)
                        acc[r, s] = acc[r, s] + acc[srows + r, s]
                    return carry

                lax.fori_loop(0, srows, sum_body, 0)
                pltpu.sync_copy(acc.at[pl.ds(0, srows)], dst)
        plsc.subcore_barrier()

        def chunk_body(ci, carry):
            base = base0 + ci * _C
            pltpu.sync_copy(comps_hbm.at[:, pl.ds(base, _C)], cv)

            def zero_body(r, carry2):
                for j in range(_HID // _L):
                    acc[r, pl.ds(j * _L, _L)] = jnp.zeros((_L,), jnp.float32)
                return carry2

            def idx_body(i, carry2):
                s = i * _L
                cx = cv[0, pl.ds(s, _L)]
                cy = cv[1, pl.ds(s, _L)]
                w = cv[2, pl.ds(s, _L)]
                h = cv[3, pl.ds(s, _L)]
                xs = cv[4, pl.ds(s, _L)]
                ys = cv[5, pl.ds(s, _L)]
                xa = ((xs - _BBOX // 2).astype(jnp.float32) * 0.5
                      ).astype(jnp.int32)
                ya = ((ys - _BBOX // 2).astype(jnp.float32) * 0.5
                      ).astype(jnp.int32)
                w2 = lax.shift_right_arithmetic(w, 1)
                h2 = lax.shift_right_arithmetic(h, 1)

                def clip(v):
                    return jnp.minimum(jnp.maximum(v, 0), _BBOX)

                sl = pl.ds(s, _L)
                idxv[0, 0, sl] = w
                idxv[1, 0, sl] = h + _TPAD
                idxv[2, 0, sl] = cx + 2 * _TPAD
                idxv[3, 0, sl] = cy + 3 * _TPAD
                idxv[4, 0, sl] = xs + 4 * _TPAD
                idxv[5, 0, sl] = ys + 5 * _TPAD
                idxv[6, 0, sl] = clip(cx - w2 - xa) + 6 * _TPAD
                idxv[7, 0, sl] = clip(cy - h2 - ya) + 7 * _TPAD
                idxv[8, 0, sl] = clip(cx + w2 - xa) + 8 * _TPAD
                idxv[9, 0, sl] = clip(cy + h2 + ya) + 9 * _TPAD
                idxv[10, 0, sl] = clip(cx + w2 + xa) + 10 * _TPAD
                idxv[11, 0, sl] = clip(cx - w2 + xa) + 11 * _TPAD
                return carry2

            lax.fori_loop(0, _C, zero_body, 0)
            lax.fori_loop(0, _C // _L, idx_body, 0)

            def fire_body(t, carry2):
                pltpu.async_copy(stab.at[idxv.at[t, 0]], acc, gsem, add=True)
                return carry2

            def drain_body(t, carry2):
                pltpu.make_async_copy(stab.at[idxv.at[t, 0]], acc, gsem
                                      ).wait()
                return carry2

            lax.fori_loop(0, _NT, fire_body, 0)
            lax.fori_loop(0, _NT, drain_body, 0)
            pltpu.sync_copy(acc, out_hbm.at[pl.ds(base, _C)])
            return carry

        lax.fori_loop(0, n_chunks, chunk_body, 0)

    return k


def kernel(boxes, tables):
    b, s, _ = boxes.shape
    n_rows = b * s
    comps = boxes.astype(jnp.int32).reshape(n_rows, 6).T
    ctab = jnp.pad(tables, ((0, 0), (0, _TPAD - _VOCAB), (0, 0))
                   ).reshape(14 * _TPAD, _HID)
    out = _build(n_rows)(comps, ctab)
    return out.reshape(b, s, _HID)
